# Initial kernel scaffold; baseline (speedup 1.0000x reference)
import jax
import jax.numpy as jnp
from jax.experimental import pallas as pl


def kernel(x_dict, task_ids, proj_W, proj_b, pln_g, pln_b, cls_token, pos_embed, ln1_g, ln1_b, Wq, Wk, Wv, bq, bk, bv, Wo, bo, ln2_g, ln2_b, task_embed, gate_W, gate_b, eW1, eb1, eW2, eb2, uW1, ub1, uW2, ub2, fln_g, fln_b, head_W, head_b):
    raise NotImplementedError("write your pallas kernel here")


# speedup vs baseline: 1.2920x; 1.2920x over previous
import functools

import numpy as np
import jax
import jax.numpy as jnp
from jax import lax
from jax.experimental import pallas as pl
from jax.experimental.pallas import tpu as pltpu

B = 32
C = 8
L_IN = 7500
SEGS = 30
SEG_LEN = 250
NFFT = 256
HOP = 128
FRAMES = 2
NFREQ = NFFT // 2 + 1
FLAT = FRAMES * NFREQ
D = 128
DFF = 512
E = 8
K = 2
T = 5
H = 8
HD = D // H
NL = 2
NTOK = C * SEGS + 1
NPAD = 256

_PREC = lax.Precision.HIGHEST


def _build_stft_basis():
    pos = np.arange(FRAMES)[:, None] * HOP + np.arange(NFFT)[None, :] - NFFT // 2
    j = np.abs(pos)
    j = np.where(j > SEG_LEN - 1, 2 * (SEG_LEN - 1) - j, j)
    ang = 2.0 * np.pi * np.outer(np.arange(NFFT), np.arange(NFREQ)) / NFFT
    cosb = np.cos(ang)
    sinb = np.sin(ang)
    a_cos = np.zeros((SEG_LEN, FLAT), np.float64)
    a_sin = np.zeros((SEG_LEN, FLAT), np.float64)
    for f in range(FRAMES):
        p = np.zeros((SEG_LEN, NFFT), np.float64)
        np.add.at(p, (j[f], np.arange(NFFT)), 1.0)
        a_cos[:, f::FRAMES] = p @ cosb
        a_sin[:, f::FRAMES] = p @ sinb
    return a_cos.astype(np.float32), a_sin.astype(np.float32)


_A_COS, _A_SIN = _build_stft_basis()


def _dot(a, b):
    return jnp.dot(a, b, preferred_element_type=jnp.float32, precision=_PREC)


def _ln(x, g, b, eps=1e-5):
    m = jnp.mean(x, axis=-1, keepdims=True)
    v = jnp.mean((x - m) ** 2, axis=-1, keepdims=True)
    return (x - m) * lax.rsqrt(v + eps) * g + b


def _gelu(x):
    return 0.5 * x * (1.0 + lax.erf(x * np.float32(1.0 / np.sqrt(2.0))))


def _tokenizer_body(x_ref, acos_ref, asin_ref, pw_ref, pb_ref, g_ref, b_ref,
                    cls_ref, pos_ref, out_ref):
    x = x_ref[0]
    re = _dot(x, acos_ref[...])
    im = _dot(x, asin_ref[...])
    mag = jnp.sqrt(re * re + im * im)
    t = _dot(mag, pw_ref[...]) + pb_ref[...]
    t = _ln(t, g_ref[...], b_ref[...])
    t = _gelu(t)
    t = t + pos_ref[...]
    full = jnp.concatenate(
        [cls_ref[...], t, jnp.zeros((NPAD - NTOK, D), jnp.float32)], axis=0)
    out_ref[0] = full


def _tokenizer(xseg, proj_W, proj_b, pln_g, pln_b, cls_pos0, pos_rest):
    return pl.pallas_call(
        _tokenizer_body,
        grid=(B,),
        in_specs=[
            pl.BlockSpec((1, C * SEGS, SEG_LEN), lambda i: (i, 0, 0)),
            pl.BlockSpec((SEG_LEN, FLAT), lambda i: (0, 0)),
            pl.BlockSpec((SEG_LEN, FLAT), lambda i: (0, 0)),
            pl.BlockSpec((FLAT, D), lambda i: (0, 0)),
            pl.BlockSpec((1, D), lambda i: (0, 0)),
            pl.BlockSpec((1, D), lambda i: (0, 0)),
            pl.BlockSpec((1, D), lambda i: (0, 0)),
            pl.BlockSpec((1, D), lambda i: (0, 0)),
            pl.BlockSpec((C * SEGS, D), lambda i: (0, 0)),
        ],
        out_specs=pl.BlockSpec((1, NPAD, D), lambda i: (i, 0, 0)),
        out_shape=jax.ShapeDtypeStruct((B, NPAD, D), jnp.float32),
    )(xseg, _A_COS, _A_SIN, proj_W, proj_b.reshape(1, D), pln_g.reshape(1, D),
      pln_b.reshape(1, D), cls_pos0, pos_rest)


def _attn_body(h_ref, g_ref, b_ref, wq_ref, wk_ref, wv_ref, bq_ref, bk_ref,
               bv_ref, wo_ref, bo_ref, out_ref):
    h = h_ref[0]
    h2 = _ln(h, g_ref[...], b_ref[...])
    q = _dot(h2, wq_ref[...]) + bq_ref[...]
    k = _dot(h2, wk_ref[...]) + bk_ref[...]
    v = _dot(h2, wv_ref[...]) + bv_ref[...]
    col = lax.broadcasted_iota(jnp.int32, (NPAD, NPAD), 1)
    kmask = col < NTOK
    scale = np.float32(1.0 / np.sqrt(HD))
    outs = []
    for hh in range(H):
        sl = slice(hh * HD, (hh + 1) * HD)
        s = lax.dot_general(q[:, sl], k[:, sl], (((1,), (1,)), ((), ())),
                            preferred_element_type=jnp.float32,
                            precision=_PREC) * scale
        s = jnp.where(kmask, s, np.float32(-1e30))
        s = s - jnp.max(s, axis=-1, keepdims=True)
        es = jnp.exp(s)
        att = es / jnp.sum(es, axis=-1, keepdims=True)
        outs.append(_dot(att, v[:, sl]))
    o = jnp.concatenate(outs, axis=1)
    out_ref[0] = h + _dot(o, wo_ref[...]) + bo_ref[...]


def _attention(h, ln_g, ln_b, wq, wk, wv, bq, bk, bv, wo, bo):
    vec = lambda: pl.BlockSpec((1, D), lambda i: (0, 0))
    mat = lambda: pl.BlockSpec((D, D), lambda i: (0, 0))
    return pl.pallas_call(
        _attn_body,
        grid=(B,),
        in_specs=[pl.BlockSpec((1, NPAD, D), lambda i: (i, 0, 0)),
                  vec(), vec(), mat(), mat(), mat(), vec(), vec(), vec(),
                  mat(), vec()],
        out_specs=pl.BlockSpec((1, NPAD, D), lambda i: (i, 0, 0)),
        out_shape=jax.ShapeDtypeStruct((B, NPAD, D), jnp.float32),
    )(h, ln_g.reshape(1, D), ln_b.reshape(1, D), wq, wk, wv,
      bq.reshape(1, D), bk.reshape(1, D), bv.reshape(1, D), wo,
      bo.reshape(1, D))


def _moe_body(ids_ref, h_ref, g_ref, b_ref, gwh_ref, gbt_ref, ew1_ref, eb1_ref,
              ew2_ref, eb2_ref, uw1_ref, ub1_ref, uw2_ref, ub2_ref,
              out_ref, logits_ref):
    h = h_ref[0]
    h2 = _ln(h, g_ref[...], b_ref[...])
    logits = _dot(h2, gwh_ref[...]) + gbt_ref[0]
    m1 = jnp.max(logits, axis=-1, keepdims=True)
    l2 = jnp.where(logits == m1, np.float32(-1e30), logits)
    m2 = jnp.max(l2, axis=-1, keepdims=True)
    keep = logits >= m2
    ex = jnp.where(keep, jnp.exp(logits - m1), 0.0)
    denom = jnp.sum(ex, axis=-1, keepdims=True)
    gates = ex / denom
    omega = 1.0 - 1.0 / denom
    acc = jnp.zeros((NPAD, D), jnp.float32)
    for ei in range(E):
        t1 = _gelu(_dot(h2, ew1_ref[ei]) + eb1_ref[ei])
        t2 = _dot(t1, ew2_ref[ei]) + eb2_ref[ei]
        acc = acc + gates[:, ei:ei + 1] * t2
    u = _gelu(_dot(h2, uw1_ref[...]) + ub1_ref[...])
    u = _dot(u, uw2_ref[...]) + ub2_ref[...]
    out_ref[0] = h + acc + omega * u
    logits_ref[0] = logits


def _moe(h, task_ids, ln_g, ln_b, gwh, gbt, ew1, eb1, ew2, eb2,
         uw1, ub1, uw2, ub2):
    vec = lambda: pl.BlockSpec((1, D), lambda i, ids: (0, 0))
    grid_spec = pltpu.PrefetchScalarGridSpec(
        num_scalar_prefetch=1,
        grid=(B,),
        in_specs=[
            pl.BlockSpec((1, NPAD, D), lambda i, ids: (i, 0, 0)),
            vec(), vec(),
            pl.BlockSpec((D, E), lambda i, ids: (0, 0)),
            pl.BlockSpec((1, 1, E), lambda i, ids: (ids[i], 0, 0)),
            pl.BlockSpec((E, D, DFF), lambda i, ids: (0, 0, 0)),
            pl.BlockSpec((E, DFF), lambda i, ids: (0, 0)),
            pl.BlockSpec((E, DFF, D), lambda i, ids: (0, 0, 0)),
            pl.BlockSpec((E, D), lambda i, ids: (0, 0)),
            pl.BlockSpec((D, DFF), lambda i, ids: (0, 0)),
            pl.BlockSpec((1, DFF), lambda i, ids: (0, 0)),
            pl.BlockSpec((DFF, D), lambda i, ids: (0, 0)),
            vec(),
        ],
        out_specs=[pl.BlockSpec((1, NPAD, D), lambda i, ids: (i, 0, 0)),
                   pl.BlockSpec((1, NPAD, E), lambda i, ids: (i, 0, 0))],
    )
    return pl.pallas_call(
        _moe_body,
        grid_spec=grid_spec,
        out_shape=[jax.ShapeDtypeStruct((B, NPAD, D), jnp.float32),
                   jax.ShapeDtypeStruct((B, NPAD, E), jnp.float32)],
    )(task_ids, h, ln_g.reshape(1, D), ln_b.reshape(1, D), gwh,
      gbt.reshape(T, 1, E),
      ew1, eb1, ew2, eb2, uw1, ub1.reshape(1, DFF), uw2, ub2.reshape(1, D))


def _head_body(ids_ref, h_ref, g_ref, b_ref, hw_ref, hb_ref, out_ref):
    c = h_ref[0]
    c = _ln(c, g_ref[...], b_ref[...])
    out_ref[0] = _dot(c, hw_ref[0]) + hb_ref[0]


def _head(cls_rows, task_ids, fln_g, fln_b, head_W, head_b):
    grid_spec = pltpu.PrefetchScalarGridSpec(
        num_scalar_prefetch=1,
        grid=(B,),
        in_specs=[
            pl.BlockSpec((1, 1, D), lambda i, ids: (i, 0, 0)),
            pl.BlockSpec((1, D), lambda i, ids: (0, 0)),
            pl.BlockSpec((1, D), lambda i, ids: (0, 0)),
            pl.BlockSpec((1, D, 2), lambda i, ids: (ids[i], 0, 0)),
            pl.BlockSpec((1, 1, 2), lambda i, ids: (ids[i], 0, 0)),
        ],
        out_specs=pl.BlockSpec((1, 1, 2), lambda i, ids: (i, 0, 0)),
    )
    return pl.pallas_call(
        _head_body,
        grid_spec=grid_spec,
        out_shape=jax.ShapeDtypeStruct((B, 1, 2), jnp.float32),
    )(task_ids, cls_rows, fln_g.reshape(1, D), fln_b.reshape(1, D), head_W,
      head_b.reshape(T, 1, 2)).reshape(B, 2)


def kernel(x_dict, task_ids, proj_W, proj_b, pln_g, pln_b, cls_token,
           pos_embed, ln1_g, ln1_b, Wq, Wk, Wv, bq, bk, bv, Wo, bo, ln2_g,
           ln2_b, task_embed, gate_W, gate_b, eW1, eb1, eW2, eb2, uW1, ub1,
           uW2, ub2, fln_g, fln_b, head_W, head_b):
    task_ids = task_ids.astype(jnp.int32)
    xseg = x_dict.reshape(B, C * SEGS, SEG_LEN)
    cls_pos0 = (cls_token.reshape(1, D) + pos_embed[0, 0].reshape(1, D))
    pos_rest = pos_embed[0, 1:NTOK]

    h = _tokenizer(xseg, proj_W, proj_b, pln_g, pln_b, cls_pos0, pos_rest)

    router = []
    for l in range(NL):
        gwh = gate_W[l][:D]
        gbt = task_embed[l] @ gate_W[l][D:] + gate_b[l][None, :]
        h = _attention(h, ln1_g[l], ln1_b[l], Wq[l], Wk[l], Wv[l], bq[l],
                       bk[l], bv[l], Wo[l], bo[l])
        h, logits = _moe(h, task_ids, ln2_g[l], ln2_b[l], gwh, gbt, eW1[l],
                         eb1[l], eW2[l], eb2[l], uW1[l], ub1[l], uW2[l],
                         ub2[l])
        router.append(logits[:, :NTOK, :])

    task_logits = _head(h[:, :1, :], task_ids, fln_g, fln_b, head_W, head_b)
    final_router = jnp.stack(router, axis=1).reshape(-1, E)
    return (task_logits, final_router)


# speedup vs baseline: 2.9636x; 2.2938x over previous
import functools

import numpy as np
import jax
import jax.numpy as jnp
from jax import lax
from jax.experimental import pallas as pl
from jax.experimental.pallas import tpu as pltpu

B = 32
C = 8
L_IN = 7500
SEGS = 30
SEG_LEN = 250
NFFT = 256
HOP = 128
FRAMES = 2
NFREQ = NFFT // 2 + 1
FLAT = FRAMES * NFREQ
D = 128
DFF = 512
E = 8
K = 2
T = 5
H = 8
HD = D // H
NL = 2
NTOK = C * SEGS + 1
NPAD = 256

_PREC = lax.Precision.HIGHEST
_PREC_FAST = lax.Precision.DEFAULT


def _build_stft_basis():
    pos = np.arange(FRAMES)[:, None] * HOP + np.arange(NFFT)[None, :] - NFFT // 2
    j = np.abs(pos)
    j = np.where(j > SEG_LEN - 1, 2 * (SEG_LEN - 1) - j, j)
    ang = 2.0 * np.pi * np.outer(np.arange(NFFT), np.arange(NFREQ)) / NFFT
    cosb = np.cos(ang)
    sinb = np.sin(ang)
    a_cos = np.zeros((SEG_LEN, FLAT), np.float64)
    a_sin = np.zeros((SEG_LEN, FLAT), np.float64)
    for f in range(FRAMES):
        p = np.zeros((SEG_LEN, NFFT), np.float64)
        np.add.at(p, (j[f], np.arange(NFFT)), 1.0)
        a_cos[:, f::FRAMES] = p @ cosb
        a_sin[:, f::FRAMES] = p @ sinb
    return a_cos.astype(np.float32), a_sin.astype(np.float32)


_A_COS, _A_SIN = _build_stft_basis()


def _dot(a, b, prec=_PREC):
    return jnp.dot(a, b, preferred_element_type=jnp.float32, precision=prec)


def _ln(x, g, b, eps=1e-5):
    m = jnp.mean(x, axis=-1, keepdims=True)
    v = jnp.mean((x - m) ** 2, axis=-1, keepdims=True)
    return (x - m) * lax.rsqrt(v + eps) * g + b


def _gelu(x):
    return 0.5 * x * (1.0 + lax.erf(x * np.float32(1.0 / np.sqrt(2.0))))


def _tokenizer_body(x_ref, acos_ref, asin_ref, pw_ref, pb_ref, g_ref, b_ref,
                    cls_ref, pos_ref, out_ref):
    x = x_ref[0]
    re = _dot(x, acos_ref[...])
    im = _dot(x, asin_ref[...])
    mag = jnp.sqrt(re * re + im * im)
    t = _dot(mag, pw_ref[...]) + pb_ref[...]
    t = _ln(t, g_ref[...], b_ref[...])
    t = _gelu(t)
    t = t + pos_ref[...]
    full = jnp.concatenate(
        [cls_ref[...], t, jnp.zeros((NPAD - NTOK, D), jnp.float32)], axis=0)
    out_ref[0] = full


def _tokenizer(xseg, proj_W, proj_b, pln_g, pln_b, cls_pos0, pos_rest):
    return pl.pallas_call(
        _tokenizer_body,
        grid=(B,),
        in_specs=[
            pl.BlockSpec((1, C * SEGS, SEG_LEN), lambda i: (i, 0, 0)),
            pl.BlockSpec((SEG_LEN, FLAT), lambda i: (0, 0)),
            pl.BlockSpec((SEG_LEN, FLAT), lambda i: (0, 0)),
            pl.BlockSpec((FLAT, D), lambda i: (0, 0)),
            pl.BlockSpec((1, D), lambda i: (0, 0)),
            pl.BlockSpec((1, D), lambda i: (0, 0)),
            pl.BlockSpec((1, D), lambda i: (0, 0)),
            pl.BlockSpec((1, D), lambda i: (0, 0)),
            pl.BlockSpec((C * SEGS, D), lambda i: (0, 0)),
        ],
        out_specs=pl.BlockSpec((1, NPAD, D), lambda i: (i, 0, 0)),
        out_shape=jax.ShapeDtypeStruct((B, NPAD, D), jnp.float32),
    )(xseg, _A_COS, _A_SIN, proj_W, proj_b.reshape(1, D), pln_g.reshape(1, D),
      pln_b.reshape(1, D), cls_pos0, pos_rest)


def _attn_body(h_ref, g_ref, b_ref, wq_ref, wk_ref, wv_ref, bq_ref, bk_ref,
               bv_ref, wo_ref, bo_ref, out_ref):
    h = h_ref[0]
    h2 = _ln(h, g_ref[...], b_ref[...])
    q = _dot(h2, wq_ref[...], _PREC_FAST) + bq_ref[...]
    k = _dot(h2, wk_ref[...], _PREC_FAST) + bk_ref[...]
    v = _dot(h2, wv_ref[...], _PREC_FAST) + bv_ref[...]
    col = lax.broadcasted_iota(jnp.int32, (NPAD, NPAD), 1)
    kmask = col < NTOK
    scale = np.float32(1.0 / np.sqrt(HD))
    outs = []
    for hh in range(H):
        sl = slice(hh * HD, (hh + 1) * HD)
        s = lax.dot_general(q[:, sl], k[:, sl], (((1,), (1,)), ((), ())),
                            preferred_element_type=jnp.float32,
                            precision=_PREC_FAST) * scale
        s = jnp.where(kmask, s, np.float32(-1e30))
        s = s - jnp.max(s, axis=-1, keepdims=True)
        es = jnp.exp(s)
        att = es / jnp.sum(es, axis=-1, keepdims=True)
        outs.append(_dot(att, v[:, sl], _PREC_FAST))
    o = jnp.concatenate(outs, axis=1)
    out_ref[0] = h + _dot(o, wo_ref[...], _PREC_FAST) + bo_ref[...]


def _attention(h, ln_g, ln_b, wq, wk, wv, bq, bk, bv, wo, bo):
    vec = lambda: pl.BlockSpec((1, D), lambda i: (0, 0))
    mat = lambda: pl.BlockSpec((D, D), lambda i: (0, 0))
    return pl.pallas_call(
        _attn_body,
        grid=(B,),
        in_specs=[pl.BlockSpec((1, NPAD, D), lambda i: (i, 0, 0)),
                  vec(), vec(), mat(), mat(), mat(), vec(), vec(), vec(),
                  mat(), vec()],
        out_specs=pl.BlockSpec((1, NPAD, D), lambda i: (i, 0, 0)),
        out_shape=jax.ShapeDtypeStruct((B, NPAD, D), jnp.float32),
    )(h, ln_g.reshape(1, D), ln_b.reshape(1, D), wq, wk, wv,
      bq.reshape(1, D), bk.reshape(1, D), bv.reshape(1, D), wo,
      bo.reshape(1, D))


def _moe_body(ids_ref, h_ref, g_ref, b_ref, gwh_ref, gbt_ref, ew1_ref, eb1_ref,
              ew2_ref, eb2_ref, uw1_ref, ub1_ref, uw2_ref, ub2_ref,
              out_ref, logits_ref):
    h = h_ref[0]
    h2 = _ln(h, g_ref[...], b_ref[...])
    logits = _dot(h2, gwh_ref[...]) + gbt_ref[0]
    m1 = jnp.max(logits, axis=-1, keepdims=True)
    l2 = jnp.where(logits == m1, np.float32(-1e30), logits)
    m2 = jnp.max(l2, axis=-1, keepdims=True)
    keep = logits >= m2
    ex = jnp.where(keep, jnp.exp(logits - m1), 0.0)
    denom = jnp.sum(ex, axis=-1, keepdims=True)
    gates = ex / denom
    omega = 1.0 - 1.0 / denom
    acc = jnp.zeros((NPAD, D), jnp.float32)
    for ei in range(E):
        t1 = _gelu(_dot(h2, ew1_ref[ei], _PREC_FAST) + eb1_ref[ei])
        t2 = _dot(t1, ew2_ref[ei], _PREC_FAST) + eb2_ref[ei]
        acc = acc + gates[:, ei:ei + 1] * t2
    u = _gelu(_dot(h2, uw1_ref[...], _PREC_FAST) + ub1_ref[...])
    u = _dot(u, uw2_ref[...], _PREC_FAST) + ub2_ref[...]
    out_ref[0] = h + acc + omega * u
    logits_ref[0] = logits


def _moe(h, task_ids, ln_g, ln_b, gwh, gbt, ew1, eb1, ew2, eb2,
         uw1, ub1, uw2, ub2):
    vec = lambda: pl.BlockSpec((1, D), lambda i, ids: (0, 0))
    grid_spec = pltpu.PrefetchScalarGridSpec(
        num_scalar_prefetch=1,
        grid=(B,),
        in_specs=[
            pl.BlockSpec((1, NPAD, D), lambda i, ids: (i, 0, 0)),
            vec(), vec(),
            pl.BlockSpec((D, E), lambda i, ids: (0, 0)),
            pl.BlockSpec((1, 1, E), lambda i, ids: (ids[i], 0, 0)),
            pl.BlockSpec((E, D, DFF), lambda i, ids: (0, 0, 0)),
            pl.BlockSpec((E, DFF), lambda i, ids: (0, 0)),
            pl.BlockSpec((E, DFF, D), lambda i, ids: (0, 0, 0)),
            pl.BlockSpec((E, D), lambda i, ids: (0, 0)),
            pl.BlockSpec((D, DFF), lambda i, ids: (0, 0)),
            pl.BlockSpec((1, DFF), lambda i, ids: (0, 0)),
            pl.BlockSpec((DFF, D), lambda i, ids: (0, 0)),
            vec(),
        ],
        out_specs=[pl.BlockSpec((1, NPAD, D), lambda i, ids: (i, 0, 0)),
                   pl.BlockSpec((1, NPAD, E), lambda i, ids: (i, 0, 0))],
    )
    return pl.pallas_call(
        _moe_body,
        grid_spec=grid_spec,
        out_shape=[jax.ShapeDtypeStruct((B, NPAD, D), jnp.float32),
                   jax.ShapeDtypeStruct((B, NPAD, E), jnp.float32)],
    )(task_ids, h, ln_g.reshape(1, D), ln_b.reshape(1, D), gwh,
      gbt.reshape(T, 1, E),
      ew1, eb1, ew2, eb2, uw1, ub1.reshape(1, DFF), uw2, ub2.reshape(1, D))


def _head_body(ids_ref, h_ref, g_ref, b_ref, hw_ref, hb_ref, out_ref):
    c = h_ref[0]
    c = _ln(c, g_ref[...], b_ref[...])
    out_ref[0] = _dot(c, hw_ref[0]) + hb_ref[0]


def _head(cls_rows, task_ids, fln_g, fln_b, head_W, head_b):
    grid_spec = pltpu.PrefetchScalarGridSpec(
        num_scalar_prefetch=1,
        grid=(B,),
        in_specs=[
            pl.BlockSpec((1, 1, D), lambda i, ids: (i, 0, 0)),
            pl.BlockSpec((1, D), lambda i, ids: (0, 0)),
            pl.BlockSpec((1, D), lambda i, ids: (0, 0)),
            pl.BlockSpec((1, D, 2), lambda i, ids: (ids[i], 0, 0)),
            pl.BlockSpec((1, 1, 2), lambda i, ids: (ids[i], 0, 0)),
        ],
        out_specs=pl.BlockSpec((1, 1, 2), lambda i, ids: (i, 0, 0)),
    )
    return pl.pallas_call(
        _head_body,
        grid_spec=grid_spec,
        out_shape=jax.ShapeDtypeStruct((B, 1, 2), jnp.float32),
    )(task_ids, cls_rows, fln_g.reshape(1, D), fln_b.reshape(1, D), head_W,
      head_b.reshape(T, 1, 2)).reshape(B, 2)


def kernel(x_dict, task_ids, proj_W, proj_b, pln_g, pln_b, cls_token,
           pos_embed, ln1_g, ln1_b, Wq, Wk, Wv, bq, bk, bv, Wo, bo, ln2_g,
           ln2_b, task_embed, gate_W, gate_b, eW1, eb1, eW2, eb2, uW1, ub1,
           uW2, ub2, fln_g, fln_b, head_W, head_b):
    task_ids = task_ids.astype(jnp.int32)
    xseg = x_dict.reshape(B, C * SEGS, SEG_LEN)
    cls_pos0 = (cls_token.reshape(1, D) + pos_embed[0, 0].reshape(1, D))
    pos_rest = pos_embed[0, 1:NTOK]

    h = _tokenizer(xseg, proj_W, proj_b, pln_g, pln_b, cls_pos0, pos_rest)

    router = []
    for l in range(NL):
        gwh = gate_W[l][:D]
        gbt = task_embed[l] @ gate_W[l][D:] + gate_b[l][None, :]
        h = _attention(h, ln1_g[l], ln1_b[l], Wq[l], Wk[l], Wv[l], bq[l],
                       bk[l], bv[l], Wo[l], bo[l])
        h, logits = _moe(h, task_ids, ln2_g[l], ln2_b[l], gwh, gbt, eW1[l],
                         eb1[l], eW2[l], eb2[l], uW1[l], ub1[l], uW2[l],
                         ub2[l])
        router.append(logits[:, :NTOK, :])

    task_logits = _head(h[:, :1, :], task_ids, fln_g, fln_b, head_W, head_b)
    final_router = jnp.stack(router, axis=1).reshape(-1, E)
    return (task_logits, final_router)


# speedup vs baseline: 3.0566x; 1.0314x over previous
import numpy as np
import jax
import jax.numpy as jnp
from jax import lax
from jax.experimental import pallas as pl
from jax.experimental.pallas import tpu as pltpu

B = 32
C = 8
SEGS = 30
SEG_LEN = 250
NFFT = 256
HOP = 128
FRAMES = 2
NFREQ = NFFT // 2 + 1
FLAT = FRAMES * NFREQ
D = 128
DFF = 512
E = 8
T = 5
H = 8
HD = D // H
NL = 2
NTOK = C * SEGS + 1
NPAD = 256

_PREC = lax.Precision.HIGHEST
_PREC_FAST = lax.Precision.DEFAULT


def _build_stft_basis():
    pos = np.arange(FRAMES)[:, None] * HOP + np.arange(NFFT)[None, :] - NFFT // 2
    j = np.abs(pos)
    j = np.where(j > SEG_LEN - 1, 2 * (SEG_LEN - 1) - j, j)
    ang = 2.0 * np.pi * np.outer(np.arange(NFFT), np.arange(NFREQ)) / NFFT
    cosb = np.cos(ang)
    sinb = np.sin(ang)
    a_cos = np.zeros((SEG_LEN, FLAT), np.float64)
    a_sin = np.zeros((SEG_LEN, FLAT), np.float64)
    for f in range(FRAMES):
        p = np.zeros((SEG_LEN, NFFT), np.float64)
        np.add.at(p, (j[f], np.arange(NFFT)), 1.0)
        a_cos[:, f::FRAMES] = p @ cosb
        a_sin[:, f::FRAMES] = p @ sinb
    return a_cos.astype(np.float32), a_sin.astype(np.float32)


_A_COS, _A_SIN = _build_stft_basis()


def _dot(a, b, prec=_PREC):
    return jnp.dot(a, b, preferred_element_type=jnp.float32, precision=prec)


def _ln(x, g, b, eps=1e-5):
    m = jnp.mean(x, axis=-1, keepdims=True)
    v = jnp.mean((x - m) ** 2, axis=-1, keepdims=True)
    return (x - m) * lax.rsqrt(v + eps) * g + b


def _gelu(x):
    return 0.5 * x * (1.0 + lax.erf(x * np.float32(1.0 / np.sqrt(2.0))))


def _attn(h, g, b, wq, wk, wv, bq, bk, bv, wo, bo, kmask):
    h2 = _ln(h, g, b)
    q = _dot(h2, wq, _PREC_FAST) + bq
    k = _dot(h2, wk, _PREC_FAST) + bk
    v = _dot(h2, wv, _PREC_FAST) + bv
    scale = np.float32(1.0 / np.sqrt(HD))
    outs = []
    for hh in range(H):
        sl = slice(hh * HD, (hh + 1) * HD)
        s = lax.dot_general(q[:, sl], k[:, sl], (((1,), (1,)), ((), ())),
                            preferred_element_type=jnp.float32,
                            precision=_PREC_FAST) * scale
        s = jnp.where(kmask, s, np.float32(-1e30))
        es = jnp.exp(s - jnp.max(s, axis=-1, keepdims=True))
        inv = 1.0 / jnp.sum(es, axis=-1, keepdims=True)
        outs.append(_dot(es, v[:, sl], _PREC_FAST) * inv)
    o = jnp.concatenate(outs, axis=1)
    return h + _dot(o, wo, _PREC_FAST) + bo


def _moe(h, g, b, gwh, gbt, ew1, eb1, ew2, eb2, uw1, ub1, uw2, ub2):
    h2 = _ln(h, g, b)
    logits = _dot(h2, gwh) + gbt
    m1 = jnp.max(logits, axis=-1, keepdims=True)
    l2 = jnp.where(logits == m1, np.float32(-1e30), logits)
    m2 = jnp.max(l2, axis=-1, keepdims=True)
    keep = logits >= m2
    ex = jnp.where(keep, jnp.exp(logits - m1), 0.0)
    denom = jnp.sum(ex, axis=-1, keepdims=True)
    gates = ex / denom
    omega = 1.0 - 1.0 / denom
    acc = jnp.zeros((NPAD, D), jnp.float32)
    for ei in range(E):
        t1 = _gelu(_dot(h2, ew1[ei], _PREC_FAST) + eb1[ei])
        t2 = _dot(t1, ew2[ei], _PREC_FAST) + eb2[ei]
        acc = acc + gates[:, ei:ei + 1] * t2
    u = _gelu(_dot(h2, uw1, _PREC_FAST) + ub1)
    u = _dot(u, uw2, _PREC_FAST) + ub2
    return h + acc + omega * u, logits


def _fwd_body(ids_ref, x_ref, acos_ref, asin_ref, pw_ref, pb_ref, png_ref,
              pnb_ref, clsp_ref, pos_ref, ln1g_ref, ln1b_ref, wq_ref, wk_ref,
              wv_ref, bq_ref, bk_ref, bv_ref, wo_ref, bo_ref, ln2g_ref,
              ln2b_ref, gwh_ref, gbt_ref, ew1_ref, eb1_ref, ew2_ref, eb2_ref,
              uw1_ref, ub1_ref, uw2_ref, ub2_ref, flng_ref, flnb_ref, hw_ref,
              hb_ref, logits_ref, out_ref):
    x = x_ref[0]
    re = _dot(x, acos_ref[...])
    im = _dot(x, asin_ref[...])
    mag = jnp.sqrt(re * re + im * im)
    t = _dot(mag, pw_ref[...]) + pb_ref[...]
    t = _ln(t, png_ref[...], pnb_ref[...])
    t = _gelu(t)
    t = t + pos_ref[...]
    h = jnp.concatenate(
        [clsp_ref[...], t, jnp.zeros((NPAD - NTOK, D), jnp.float32)], axis=0)

    kmask = lax.broadcasted_iota(jnp.int32, (NPAD, NPAD), 1) < NTOK
    for l in range(NL):
        h = _attn(h, ln1g_ref[l:l + 1], ln1b_ref[l:l + 1], wq_ref[l],
                  wk_ref[l], wv_ref[l], bq_ref[l:l + 1], bk_ref[l:l + 1],
                  bv_ref[l:l + 1], wo_ref[l], bo_ref[l:l + 1], kmask)
        h, logits = _moe(h, ln2g_ref[l:l + 1], ln2b_ref[l:l + 1], gwh_ref[l],
                         gbt_ref[0, l:l + 1], ew1_ref[l], eb1_ref[l],
                         ew2_ref[l], eb2_ref[l], uw1_ref[l],
                         ub1_ref[l, 0:1], uw2_ref[l], ub2_ref[l, 0:1])
        logits_ref[0, l] = logits

    c = _ln(h[0:1, :], flng_ref[...], flnb_ref[...])
    out_ref[0] = _dot(c, hw_ref[0]) + hb_ref[0]


def kernel(x_dict, task_ids, proj_W, proj_b, pln_g, pln_b, cls_token,
           pos_embed, ln1_g, ln1_b, Wq, Wk, Wv, bq, bk, bv, Wo, bo, ln2_g,
           ln2_b, task_embed, gate_W, gate_b, eW1, eb1, eW2, eb2, uW1, ub1,
           uW2, ub2, fln_g, fln_b, head_W, head_b):
    task_ids = task_ids.astype(jnp.int32)
    xseg = x_dict.reshape(B, C * SEGS, SEG_LEN)
    cls_pos0 = (cls_token.reshape(1, D) + pos_embed[0, 0].reshape(1, D))
    pos_rest = pos_embed[0, 1:NTOK]

    gwh = gate_W[:, :D, :]
    gbt = (jnp.einsum('lte,leo->tlo', task_embed, gate_W[:, D:, :])
           + gate_b[None, :, :])

    full = lambda shp: pl.BlockSpec(shp, lambda i, ids: (0,) * len(shp))
    grid_spec = pltpu.PrefetchScalarGridSpec(
        num_scalar_prefetch=1,
        grid=(B,),
        in_specs=[
            pl.BlockSpec((1, C * SEGS, SEG_LEN), lambda i, ids: (i, 0, 0)),
            full((SEG_LEN, FLAT)), full((SEG_LEN, FLAT)),
            full((FLAT, D)), full((1, D)), full((1, D)), full((1, D)),
            full((1, D)), full((C * SEGS, D)),
            full((NL, D)), full((NL, D)),
            full((NL, D, D)), full((NL, D, D)), full((NL, D, D)),
            full((NL, D)), full((NL, D)), full((NL, D)),
            full((NL, D, D)), full((NL, D)),
            full((NL, D)), full((NL, D)),
            full((NL, D, E)),
            pl.BlockSpec((1, NL, E), lambda i, ids: (ids[i], 0, 0)),
            full((NL, E, D, DFF)), full((NL, E, DFF)),
            full((NL, E, DFF, D)), full((NL, E, D)),
            full((NL, D, DFF)), full((NL, 1, DFF)),
            full((NL, DFF, D)), full((NL, 1, D)),
            full((1, D)), full((1, D)),
            pl.BlockSpec((1, D, 2), lambda i, ids: (ids[i], 0, 0)),
            pl.BlockSpec((1, 1, 2), lambda i, ids: (ids[i], 0, 0)),
        ],
        out_specs=[
            pl.BlockSpec((1, NL, NPAD, E), lambda i, ids: (i, 0, 0, 0)),
            pl.BlockSpec((1, 1, 2), lambda i, ids: (i, 0, 0)),
        ],
    )
    logits_all, task_logits = pl.pallas_call(
        _fwd_body,
        grid_spec=grid_spec,
        out_shape=[jax.ShapeDtypeStruct((B, NL, NPAD, E), jnp.float32),
                   jax.ShapeDtypeStruct((B, 1, 2), jnp.float32)],
    )(task_ids, xseg, _A_COS, _A_SIN, proj_W, proj_b.reshape(1, D),
      pln_g.reshape(1, D), pln_b.reshape(1, D), cls_pos0, pos_rest,
      ln1_g, ln1_b, Wq, Wk, Wv, bq, bk, bv, Wo, bo, ln2_g, ln2_b,
      gwh, gbt, eW1, eb1, eW2, eb2, uW1, ub1.reshape(NL, 1, DFF),
      uW2, ub2.reshape(NL, 1, D), fln_g.reshape(1, D), fln_b.reshape(1, D),
      head_W, head_b.reshape(T, 1, 2))

    final_router = logits_all[:, :, :NTOK, :].reshape(-1, E)
    return (task_logits.reshape(B, 2), final_router)
